# baseline (speedup 1.0000x reference)
import dataclasses
import functools

import jax
import jax.numpy as jnp
from jax import lax
from jax.experimental import pallas as pl
from jax.experimental.pallas import tpu as pltpu
from jax.experimental.pallas import tpu_sc as plsc

_D = 32
_NW = 32
_CHUNK = 128
_NBUF = 4
_LANES = 16


def _sc_compiler_params(tc_tiling=False):
    cp = pltpu.CompilerParams(use_tc_tiling_on_sc=tc_tiling)
    if "needs_layout_passes" in pltpu.CompilerParams.__dataclass_fields__:
        cp = dataclasses.replace(cp, needs_layout_passes=False)
    return cp


_PB = 512


_PNB = 3


def _table_repack_sc(loc_t, tail):
    d, vocab = loc_t.shape
    n_full = vocab // _PB
    rem = vocab - n_full * _PB
    n_iters = (n_full + _NW - 1) // _NW
    out_rows_pb = _PB * d // 128
    rows_per_out = 128 // d
    mesh = plsc.VectorSubcoreMesh(core_axis_name="c", subcore_axis_name="s")

    @functools.partial(
        pl.kernel,
        out_type=jax.ShapeDtypeStruct((vocab * d // 128, 128), jnp.float32),
        mesh=mesh,
        scratch_types=[
            pltpu.VMEM((_PNB, d, _PB + 5), jnp.float32),
            pltpu.VMEM((_PNB, out_rows_pb, 128), jnp.float32),
            pltpu.SemaphoreType.DMA((_PNB,)),
            pltpu.SemaphoreType.DMA((_PNB,)),
        ],
        compiler_params=_sc_compiler_params(tc_tiling=True),
    )
    def repack(loc_hbm, tail_hbm, out_hbm, inblk, outblk, sem_i, sem_o):
        wid = lax.axis_index("s") * 2 + lax.axis_index("c")
        iota = lax.iota(jnp.int32, _LANES)

        def issue_in(b, gb):
            pltpu.async_copy(
                loc_hbm.at[:, pl.ds(gb * _PB, _PB)],
                inblk.at[b, :, pl.ds(0, _PB)], sem_i.at[b])

        def wait_in(b):
            pltpu.make_async_copy(
                loc_hbm.at[:, pl.ds(0, _PB)],
                inblk.at[b, :, pl.ds(0, _PB)], sem_i.at[b]).wait()

        def issue_out(b, gb):
            pltpu.async_copy(
                outblk.at[b],
                out_hbm.at[pl.ds(gb * out_rows_pb, out_rows_pb)],
                sem_o.at[b])

        def wait_out(b):
            pltpu.make_async_copy(
                outblk.at[b],
                out_hbm.at[pl.ds(0, out_rows_pb)], sem_o.at[b]).wait()

        def transpose_block(b, n_rows):
            @plsc.parallel_loop(0, n_rows // rows_per_out, unroll=8)
            def _(r2):
                for h in range(0, 128, _LANES):
                    jv = (h % d) + iota
                    sv = jnp.full((_LANES,), 0, jnp.int32) + (
                        rows_per_out * r2 + h // d)
                    v = plsc.load_gather(inblk.at[b], [jv, sv])
                    outblk[b, r2, pl.ds(h, _LANES)] = v

        for b in range(_PNB):
            issue_in(b, b * _NW + wid)

        @pl.loop(0, n_iters, step=_PNB)
        def _(i0):
            for b in range(_PNB):
                i = i0 + b
                gb = i * _NW + wid

                @pl.when(gb < n_full)
                def _():
                    wait_in(b)

                    @pl.when(i >= _PNB)
                    def _():
                        wait_out(b)

                    transpose_block(b, _PB)
                    issue_out(b, gb)

                nxt = (i + _PNB) * _NW + wid

                @pl.when(jnp.logical_and(gb < n_full, nxt < n_full))
                def _():
                    issue_in(b, nxt)

        for b in range(_PNB):
            @pl.when((b * _NW + wid) < n_full)
            def _():
                wait_out(b)

        if rem:
            @pl.when(wid == 0)
            def _():
                n_out = rem * d // 128
                pltpu.sync_copy(tail_hbm, outblk.at[0, pl.ds(0, n_out)])
                pltpu.sync_copy(
                    outblk.at[0, pl.ds(0, n_out)],
                    out_hbm.at[pl.ds(n_full * _PB * d // 128, n_out)])

    return repack(loc_t, tail)


def _emb_sum_sc(srcT, durT, emb_loc, emb_durT, *, seq_len, batch):
    dur_vocab = emb_durT.shape[1]
    mesh = plsc.VectorSubcoreMesh(core_axis_name="c", subcore_axis_name="s")

    @functools.partial(
        pl.kernel,
        out_type=jax.ShapeDtypeStruct((seq_len, _D, batch), jnp.float32),
        mesh=mesh,
        scratch_types=[
            pltpu.VMEM((seq_len, _CHUNK), jnp.int32),
            pltpu.VMEM((seq_len, _CHUNK), jnp.int32),
            pltpu.VMEM((_D, dur_vocab), jnp.float32),
            pltpu.VMEM((_NBUF, _CHUNK, _D), jnp.float32),
            pltpu.VMEM((_D, _CHUNK + 1), jnp.float32),
            pltpu.VMEM((_NBUF, _D, _CHUNK), jnp.float32),
            pltpu.SemaphoreType.DMA((_NBUF,)),
            pltpu.SemaphoreType.DMA((_NBUF,)),
        ],
        compiler_params=_sc_compiler_params(),
    )
    def emb_kernel(src_hbm, dur_hbm, loc_hbm, durtab_hbm, out_hbm,
                   sidx, didx, durtab, arows, apad, orows, sem_g, sem_o):
        wid = lax.axis_index("s") * 2 + lax.axis_index("c")
        col0 = wid * _CHUNK
        pltpu.sync_copy(src_hbm.at[:, pl.ds(col0, _CHUNK)], sidx)
        pltpu.sync_copy(dur_hbm.at[:, pl.ds(col0, _CHUNK)], didx)
        pltpu.sync_copy(durtab_hbm, durtab)

        def issue_gather(b, l):
            pltpu.async_copy(
                loc_hbm.at[sidx.at[l]], arows.at[b], sem_g.at[b])

        def wait_gather(b):
            pltpu.make_async_copy(
                loc_hbm.at[sidx.at[0]], arows.at[b], sem_g.at[b]).wait()

        def wait_out(b):
            pltpu.make_async_copy(
                orows.at[b], out_hbm.at[0, :, pl.ds(col0, _CHUNK)],
                sem_o.at[b]).wait()

        iota = lax.iota(jnp.int32, _LANES)

        for b in range(_NBUF):
            issue_gather(b, b)

        @pl.loop(0, seq_len, step=_NBUF)
        def _(l0):
            for b in range(_NBUF):
                l = l0 + b
                wait_gather(b)

                @pl.when(l >= _NBUF)
                def _():
                    wait_out(b)

                @plsc.parallel_loop(0, _CHUNK, step=4, unroll=4)
                def _(r0):
                    for dr in range(4):
                        r = r0 + dr
                        rsplat = jnp.full((_LANES,), 0, jnp.int32) + r
                        for jh in (0, 16):
                            v = arows[b, r, pl.ds(jh, _LANES)]
                            plsc.store_scatter(apad, [jh + iota, rsplat], v)

                @plsc.parallel_loop(0, _CHUNK, step=_LANES, unroll=4)
                def _(g0):
                    rowv = g0 + iota
                    dvec = didx[l, pl.ds(g0, _LANES)]
                    for j in range(_D):
                        jv = jnp.full((_LANES,), j, jnp.int32)
                        t = plsc.load_gather(durtab, [jv, dvec])
                        cur = plsc.load_gather(apad, [jv, rowv])
                        orows[b, j, pl.ds(g0, _LANES)] = cur + t

                pltpu.async_copy(
                    orows.at[b], out_hbm.at[l, :, pl.ds(col0, _CHUNK)],
                    sem_o.at[b])

                @pl.when(l + _NBUF < seq_len)
                def _():
                    issue_gather(b, l + _NBUF)

        for b in range(_NBUF):
            wait_out(b)

    return emb_kernel(srcT, durT, emb_loc, emb_durT)


def kernel(src, duration, emb_loc, emb_dur):
    b, l = src.shape
    vocab, d = emb_loc.shape
    n_full_rows = (vocab // _PB) * _PB
    tail = jnp.reshape(emb_loc[n_full_rows:, :], (-1, 128))
    loc_lin = _table_repack_sc(jnp.transpose(emb_loc), tail)
    out_t = _emb_sum_sc(
        jnp.transpose(src).astype(jnp.int32),
        jnp.transpose(duration).astype(jnp.int32),
        jnp.reshape(loc_lin, (vocab, d)),
        jnp.transpose(emb_dur),
        seq_len=l, batch=b)
    return jnp.transpose(out_t, (2, 0, 1))


# speedup vs baseline: 1.0012x; 1.0012x over previous
import dataclasses
import functools

import jax
import jax.numpy as jnp
from jax import lax
from jax.experimental import pallas as pl
from jax.experimental.pallas import tpu as pltpu
from jax.experimental.pallas import tpu_sc as plsc

_D = 32
_NW = 32
_CHUNK = 128
_NBUF = 4
_LANES = 16


def _sc_compiler_params(tc_tiling=False):
    cp = pltpu.CompilerParams(use_tc_tiling_on_sc=tc_tiling)
    if "needs_layout_passes" in pltpu.CompilerParams.__dataclass_fields__:
        cp = dataclasses.replace(cp, needs_layout_passes=False)
    return cp


_PB = 512


_PNB = 2


def _table_repack_sc(loc_t, tail):
    d, vocab = loc_t.shape
    n_full = vocab // _PB
    rem = vocab - n_full * _PB
    n_iters = (n_full + _NW - 1) // _NW
    out_rows_pb = _PB * d // 128
    rows_per_out = 128 // d
    mesh = plsc.VectorSubcoreMesh(core_axis_name="c", subcore_axis_name="s")

    @functools.partial(
        pl.kernel,
        out_type=jax.ShapeDtypeStruct((vocab * d // 128, 128), jnp.float32),
        mesh=mesh,
        scratch_types=[
            pltpu.VMEM((_PNB, d, _PB + 5), jnp.float32),
            pltpu.VMEM((_PNB, out_rows_pb, 128), jnp.float32),
            pltpu.SemaphoreType.DMA((_PNB,)),
            pltpu.SemaphoreType.DMA((_PNB,)),
        ],
        compiler_params=_sc_compiler_params(tc_tiling=True),
    )
    def repack(loc_hbm, tail_hbm, out_hbm, inblk, outblk, sem_i, sem_o):
        wid = lax.axis_index("s") * 2 + lax.axis_index("c")
        iota = lax.iota(jnp.int32, _LANES)

        def issue_in(b, gb):
            pltpu.async_copy(
                loc_hbm.at[:, pl.ds(gb * _PB, _PB)],
                inblk.at[b, :, pl.ds(0, _PB)], sem_i.at[b])

        def wait_in(b):
            pltpu.make_async_copy(
                loc_hbm.at[:, pl.ds(0, _PB)],
                inblk.at[b, :, pl.ds(0, _PB)], sem_i.at[b]).wait()

        def issue_out(b, gb):
            pltpu.async_copy(
                outblk.at[b],
                out_hbm.at[pl.ds(gb * out_rows_pb, out_rows_pb)],
                sem_o.at[b])

        def wait_out(b):
            pltpu.make_async_copy(
                outblk.at[b],
                out_hbm.at[pl.ds(0, out_rows_pb)], sem_o.at[b]).wait()

        def transpose_block(b, n_rows):
            @plsc.parallel_loop(0, n_rows // rows_per_out, unroll=4)
            def _(r2):
                for h in range(0, 128, _LANES):
                    jv = (h % d) + iota
                    sv = jnp.full((_LANES,), 0, jnp.int32) + (
                        rows_per_out * r2 + h // d)
                    v = plsc.load_gather(inblk.at[b], [jv, sv])
                    outblk[b, r2, pl.ds(h, _LANES)] = v

        for b in range(_PNB):
            issue_in(b, b * _NW + wid)

        @pl.loop(0, n_iters, step=_PNB)
        def _(i0):
            for b in range(_PNB):
                i = i0 + b
                gb = i * _NW + wid

                @pl.when(gb < n_full)
                def _():
                    wait_in(b)

                    @pl.when(i >= _PNB)
                    def _():
                        wait_out(b)

                    transpose_block(b, _PB)
                    issue_out(b, gb)

                nxt = (i + _PNB) * _NW + wid

                @pl.when(jnp.logical_and(gb < n_full, nxt < n_full))
                def _():
                    issue_in(b, nxt)

        for b in range(_PNB):
            @pl.when((b * _NW + wid) < n_full)
            def _():
                wait_out(b)

        if rem:
            @pl.when(wid == 0)
            def _():
                n_out = rem * d // 128
                pltpu.sync_copy(tail_hbm, outblk.at[0, pl.ds(0, n_out)])
                pltpu.sync_copy(
                    outblk.at[0, pl.ds(0, n_out)],
                    out_hbm.at[pl.ds(n_full * _PB * d // 128, n_out)])

    return repack(loc_t, tail)


def _emb_sum_sc(srcT, durT, emb_loc, emb_durT, *, seq_len, batch):
    dur_vocab = emb_durT.shape[1]
    mesh = plsc.VectorSubcoreMesh(core_axis_name="c", subcore_axis_name="s")

    @functools.partial(
        pl.kernel,
        out_type=jax.ShapeDtypeStruct((seq_len, _D, batch), jnp.float32),
        mesh=mesh,
        scratch_types=[
            pltpu.VMEM((seq_len, _CHUNK), jnp.int32),
            pltpu.VMEM((seq_len, _CHUNK), jnp.int32),
            pltpu.VMEM((_D, dur_vocab), jnp.float32),
            pltpu.VMEM((_NBUF, _CHUNK, _D), jnp.float32),
            pltpu.VMEM((_D, _CHUNK + 1), jnp.float32),
            pltpu.VMEM((_NBUF, _D, _CHUNK), jnp.float32),
            pltpu.SemaphoreType.DMA((_NBUF,)),
            pltpu.SemaphoreType.DMA((_NBUF,)),
        ],
        compiler_params=_sc_compiler_params(),
    )
    def emb_kernel(src_hbm, dur_hbm, loc_hbm, durtab_hbm, out_hbm,
                   sidx, didx, durtab, arows, apad, orows, sem_g, sem_o):
        wid = lax.axis_index("s") * 2 + lax.axis_index("c")
        col0 = wid * _CHUNK
        pltpu.sync_copy(src_hbm.at[:, pl.ds(col0, _CHUNK)], sidx)
        pltpu.sync_copy(dur_hbm.at[:, pl.ds(col0, _CHUNK)], didx)
        pltpu.sync_copy(durtab_hbm, durtab)

        def issue_gather(b, l):
            pltpu.async_copy(
                loc_hbm.at[sidx.at[l]], arows.at[b], sem_g.at[b])

        def wait_gather(b):
            pltpu.make_async_copy(
                loc_hbm.at[sidx.at[0]], arows.at[b], sem_g.at[b]).wait()

        def wait_out(b):
            pltpu.make_async_copy(
                orows.at[b], out_hbm.at[0, :, pl.ds(col0, _CHUNK)],
                sem_o.at[b]).wait()

        iota = lax.iota(jnp.int32, _LANES)

        for b in range(_NBUF):
            issue_gather(b, b)

        @pl.loop(0, seq_len, step=_NBUF)
        def _(l0):
            for b in range(_NBUF):
                l = l0 + b
                wait_gather(b)

                @pl.when(l >= _NBUF)
                def _():
                    wait_out(b)

                @plsc.parallel_loop(0, _CHUNK, step=4, unroll=4)
                def _(r0):
                    for dr in range(4):
                        r = r0 + dr
                        rsplat = jnp.full((_LANES,), 0, jnp.int32) + r
                        for jh in (0, 16):
                            v = arows[b, r, pl.ds(jh, _LANES)]
                            plsc.store_scatter(apad, [jh + iota, rsplat], v)

                @plsc.parallel_loop(0, _CHUNK, step=_LANES, unroll=4)
                def _(g0):
                    rowv = g0 + iota
                    dvec = didx[l, pl.ds(g0, _LANES)]
                    for j in range(_D):
                        jv = jnp.full((_LANES,), j, jnp.int32)
                        t = plsc.load_gather(durtab, [jv, dvec])
                        cur = plsc.load_gather(apad, [jv, rowv])
                        orows[b, j, pl.ds(g0, _LANES)] = cur + t

                pltpu.async_copy(
                    orows.at[b], out_hbm.at[l, :, pl.ds(col0, _CHUNK)],
                    sem_o.at[b])

                @pl.when(l + _NBUF < seq_len)
                def _():
                    issue_gather(b, l + _NBUF)

        for b in range(_NBUF):
            wait_out(b)

    return emb_kernel(srcT, durT, emb_loc, emb_durT)


def kernel(src, duration, emb_loc, emb_dur):
    b, l = src.shape
    vocab, d = emb_loc.shape
    n_full_rows = (vocab // _PB) * _PB
    tail = jnp.reshape(emb_loc[n_full_rows:, :], (-1, 128))
    loc_lin = _table_repack_sc(jnp.transpose(emb_loc), tail)
    out_t = _emb_sum_sc(
        jnp.transpose(src).astype(jnp.int32),
        jnp.transpose(duration).astype(jnp.int32),
        jnp.reshape(loc_lin, (vocab, d)),
        jnp.transpose(emb_dur),
        seq_len=l, batch=b)
    return jnp.transpose(out_t, (2, 0, 1))
